# baseline (speedup 1.0000x reference)
import functools

import jax
import jax.numpy as jnp
from jax import lax
from jax.experimental import pallas as pl
from jax.experimental.pallas import tpu as pltpu
from jax.experimental.pallas import tpu_sc as plsc

NC = 2
NS = 16
NW = NC * NS
L = 16

K = 32
D = 128
G = 4
GK = G * K
CH = GK // 128
NCHUNK = D // L


def _pool_body(points_hbm, idx_hbm, out_hbm, table_sh, idx_v, rows_v, out_v,
               gsem0, gsem1, osem0, osem1, *, nb, n):
    gsems = (gsem0, gsem1)
    osems = (osem0, osem1)
    sid = lax.axis_index("s")
    wid = sid * NC + lax.axis_index("c")
    row_base = wid * (nb * G)

    @pl.when(sid == 0)
    def _():
        pltpu.sync_copy(points_hbm, table_sh)

    pltpu.sync_copy(idx_hbm.at[pl.ds(wid * (nb * CH), nb * CH)], idx_v)
    plsc.subcore_barrier()

    def fire_gather(batch, buf):
        for c in range(CH):
            pltpu.async_copy(table_sh.at[idx_v.at[batch * CH + c]],
                             rows_v.at[buf, pl.ds(c * 128, 128)],
                             gsems[buf])

    def wait_gather(batch, buf):
        for c in range(CH):
            pltpu.make_async_copy(table_sh.at[idx_v.at[batch * CH + c]],
                                  rows_v.at[buf, pl.ds(c * 128, 128)],
                                  gsems[buf]).wait()

    def fire_store(batch, buf):
        pltpu.async_copy(out_v.at[buf],
                         out_hbm.at[pl.ds(row_base + batch * G, G)],
                         osems[buf])

    def wait_store(batch, buf):
        pltpu.make_async_copy(out_v.at[buf],
                              out_hbm.at[pl.ds(row_base + batch * G, G)],
                              osems[buf]).wait()

    def compute(buf):
        rv = rows_v.at[buf]
        ov = out_v.at[buf]

        UNR = 16

        def per_row(g, carry):
            r0 = g * K
            accs = tuple(rv[r0 + j, pl.ds(c * L, L)] for c in range(NCHUNK)
                         for j in (0,))

            def jstep(t, accs):
                r = r0 + t * UNR
                for j in range(UNR):
                    accs = tuple(
                        jnp.maximum(accs[c], rv[r + j, pl.ds(c * L, L)])
                        for c in range(NCHUNK))
                return accs

            accs = lax.fori_loop(1, K // UNR, jstep, jstep(0, accs))
            for c in range(NCHUNK):
                ov[g, pl.ds(c * L, L)] = accs[c]
            return carry

        lax.fori_loop(0, G, per_row, 0)

    fire_gather(0, 0)

    def two_batches(t, carry):
        for b in range(2):
            i = 2 * t + b
            nbuf = (b + 1) % 2

            @pl.when(i + 1 < nb)
            def _():
                fire_gather(i + 1, nbuf)

            wait_gather(i, b)

            @pl.when(i >= 2)
            def _():
                wait_store(i - 2, b)

            compute(b)
            fire_store(i, b)
        return carry

    lax.fori_loop(0, nb // 2, two_batches, 0)
    wait_store(nb - 2, 0)
    wait_store(nb - 1, 1)


def kernel(points, indices):
    m, k = indices.shape
    n, d = points.shape
    assert k == K and d == D

    rows_per_w = -(-m // (NW * G)) * G
    nb = rows_per_w // G
    if nb % 2:
        nb += 1
        rows_per_w += G
    m_pad = NW * rows_per_w

    idx = indices.astype(jnp.int32)
    idx = jnp.pad(idx, ((0, m_pad - m), (0, 0)))
    idx2 = idx.reshape(m_pad * K // 128, 128)

    pool = functools.partial(
        pl.kernel,
        out_type=jax.ShapeDtypeStruct((m_pad, D), jnp.float32),
        mesh=plsc.VectorSubcoreMesh(core_axis_name="c", subcore_axis_name="s"),
        scratch_types=[
            pltpu.VMEM_SHARED((n, D), jnp.float32),
            pltpu.VMEM((nb * CH, 128), jnp.int32),
            pltpu.VMEM((2, GK, D), jnp.float32),
            pltpu.VMEM((2, G, D), jnp.float32),
            pltpu.SemaphoreType.DMA,
            pltpu.SemaphoreType.DMA,
            pltpu.SemaphoreType.DMA,
            pltpu.SemaphoreType.DMA,
        ],
    )(functools.partial(_pool_body, nb=nb, n=n))

    out = pool(points, idx2)
    return out[:m]


# speedup vs baseline: 1.5780x; 1.5780x over previous
import functools

import jax
import jax.numpy as jnp
from jax import lax
from jax.experimental import pallas as pl
from jax.experimental.pallas import tpu as pltpu
from jax.experimental.pallas import tpu_sc as plsc

NC = 2
NS = 16
NW = NC * NS
L = 16

K = 32
D = 128
G = 4
GK = G * K
CH = GK // 128
NCHUNK = D // L


def _pool_body(points_hbm, idx_hbm, out_hbm, table_sh, idx_v, rows_v, out_v,
               gsem0, gsem1, osem0, osem1, *, nb, n):
    gsems = (gsem0, gsem1)
    osems = (osem0, osem1)
    sid = lax.axis_index("s")
    wid = sid * NC + lax.axis_index("c")
    row_base = wid * (nb * G)

    @pl.when(sid == 0)
    def _():
        pltpu.sync_copy(points_hbm, table_sh)

    pltpu.sync_copy(idx_hbm.at[pl.ds(wid * (nb * CH), nb * CH)], idx_v)
    plsc.subcore_barrier()

    def fire_gather(batch, buf):
        for c in range(CH):
            pltpu.async_copy(table_sh.at[idx_v.at[batch * CH + c]],
                             rows_v.at[buf, pl.ds(c * 128, 128)],
                             gsems[buf])

    def wait_gather(batch, buf):
        for c in range(CH):
            pltpu.make_async_copy(table_sh.at[idx_v.at[batch * CH + c]],
                                  rows_v.at[buf, pl.ds(c * 128, 128)],
                                  gsems[buf]).wait()

    def fire_store(batch, buf):
        pltpu.async_copy(out_v.at[buf],
                         out_hbm.at[pl.ds(row_base + batch * G, G)],
                         osems[buf])

    def wait_store(batch, buf):
        pltpu.make_async_copy(out_v.at[buf],
                              out_hbm.at[pl.ds(row_base + batch * G, G)],
                              osems[buf]).wait()

    def compute(buf):
        rv = rows_v.at[buf]
        ov = out_v.at[buf]

        UNR = 8

        neg_inf = jnp.full((L,), -jnp.inf, dtype=jnp.float32)

        def per_row(g, carry):
            r0 = g * K
            accs = (neg_inf,) * NCHUNK

            def jstep(t, accs):
                r = r0 + t * UNR
                for j in range(UNR):
                    accs = tuple(
                        jnp.maximum(accs[c], rv[r + j, pl.ds(c * L, L)])
                        for c in range(NCHUNK))
                return accs

            accs = lax.fori_loop(0, K // UNR, jstep, accs)
            for c in range(NCHUNK):
                ov[g, pl.ds(c * L, L)] = accs[c]
            return carry

        lax.fori_loop(0, G, per_row, 0)

    fire_gather(0, 0)

    def two_batches(t, carry):
        for b in range(2):
            i = 2 * t + b
            nbuf = (b + 1) % 2

            @pl.when(i + 1 < nb)
            def _():
                fire_gather(i + 1, nbuf)

            wait_gather(i, b)

            @pl.when(i >= 2)
            def _():
                wait_store(i - 2, b)

            compute(b)
            fire_store(i, b)
        return carry

    lax.fori_loop(0, nb // 2, two_batches, 0)
    wait_store(nb - 2, 0)
    wait_store(nb - 1, 1)


def kernel(points, indices):
    m, k = indices.shape
    n, d = points.shape
    assert k == K and d == D

    rows_per_w = -(-m // (NW * G)) * G
    nb = rows_per_w // G
    if nb % 2:
        nb += 1
        rows_per_w += G
    m_pad = NW * rows_per_w

    idx = indices.astype(jnp.int32)
    idx = jnp.pad(idx, ((0, m_pad - m), (0, 0)))
    idx2 = idx.reshape(m_pad * K // 128, 128)

    pool = functools.partial(
        pl.kernel,
        out_type=jax.ShapeDtypeStruct((m_pad, D), jnp.float32),
        mesh=plsc.VectorSubcoreMesh(core_axis_name="c", subcore_axis_name="s"),
        scratch_types=[
            pltpu.VMEM_SHARED((n, D), jnp.float32),
            pltpu.VMEM((nb * CH, 128), jnp.int32),
            pltpu.VMEM((2, GK, D), jnp.float32),
            pltpu.VMEM((2, G, D), jnp.float32),
            pltpu.SemaphoreType.DMA,
            pltpu.SemaphoreType.DMA,
            pltpu.SemaphoreType.DMA,
            pltpu.SemaphoreType.DMA,
        ],
    )(functools.partial(_pool_body, nb=nb, n=n))

    out = pool(points, idx2)
    return out[:m]
